# baseline (speedup 1.0000x reference)
import functools
import jax
import jax.numpy as jnp
from jax import lax
from jax.experimental import pallas as pl
from jax.experimental.pallas import tpu as pltpu
from jax.experimental.pallas import tpu_sc as plsc

T = 4096
D = 1024
H = 1024
E = 8
K = 2

BT = 1024
B = 256
LOG2B = 8
NB = 40
NBP = 48
NP = NB * B
NW = 32
CHUNK = 2 * T // NW
L = 16
TPW = T // NW


def _gating_body(x_ref, wg_ref, e01_ref, w01_ref, hist_ref):
    x = x_ref[...]
    logits = jnp.dot(x, wg_ref[...], preferred_element_type=jnp.float32)
    ids = lax.broadcasted_iota(jnp.int32, logits.shape, 1)
    m1 = jnp.max(logits, axis=1, keepdims=True)
    i1 = jnp.min(jnp.where(logits == m1, ids, E), axis=1, keepdims=True)
    masked = jnp.where(ids == i1, -jnp.inf, logits)
    m2 = jnp.max(masked, axis=1, keepdims=True)
    i2 = jnp.min(jnp.where(masked == m2, ids, E), axis=1, keepdims=True)
    s = jnp.exp(m2 - m1)
    w1g = 1.0 / (1.0 + s)
    w2g = 1.0 - w1g
    e01_ref[...] = lax.transpose(
        jnp.concatenate([i1, i2], axis=0), (1, 0)).reshape(1, 1, 2 * BT)
    w01_ref[...] = lax.transpose(
        jnp.concatenate([w1g, w2g], axis=0), (1, 0)).reshape(1, 1, 2 * BT)
    oh0 = (i1 == ids).astype(jnp.int32)
    oh1 = (i2 == ids).astype(jnp.int32)
    subs = [jnp.sum(oh0[ss * 256:(ss + 1) * 256], axis=0)
            for ss in range(BT // 256)]
    subs += [jnp.sum(oh1[ss * 256:(ss + 1) * 256], axis=0)
             for ss in range(BT // 256)]
    hist_ref[...] = jnp.concatenate(subs).reshape(1, 1, 2 * BT // 256 * E)


def _gating(x, Wg):
    nblk = T // BT
    hlane = 2 * BT // 256 * E
    return pl.pallas_call(
        _gating_body,
        grid=(nblk,),
        in_specs=[
            pl.BlockSpec((BT, D), lambda i: (i, 0)),
            pl.BlockSpec((D, E), lambda i: (0, 0)),
        ],
        out_specs=[
            pl.BlockSpec((1, 1, 2 * BT), lambda i: (i, 0, 0)),
            pl.BlockSpec((1, 1, 2 * BT), lambda i: (i, 0, 0)),
            pl.BlockSpec((1, 1, hlane), lambda i: (i, 0, 0)),
        ],
        out_shape=[
            jax.ShapeDtypeStruct((nblk, 1, 2 * BT), jnp.int32),
            jax.ShapeDtypeStruct((nblk, 1, 2 * BT), jnp.float32),
            jax.ShapeDtypeStruct((nblk, 1, hlane), jnp.int32),
        ],
        compiler_params=pltpu.CompilerParams(
            dimension_semantics=("parallel",),
        ),
    )(x, Wg)


def _vgather(vec, idx):
    return lax.gather(
        vec, idx[:, None],
        lax.GatherDimensionNumbers(
            offset_dims=(), collapsed_slice_dims=(0,), start_index_map=(0,)),
        (1,), mode=lax.GatherScatterMode.PROMISE_IN_BOUNDS)


def _splat(vec, e):
    return _vgather(vec, jnp.full((L,), e, jnp.int32))


RCH = 32
NCH = CHUNK // RCH


def _routing_body(e01_hbm, hist_hbm, x_hbm,
                  xs_hbm, pos_hbm, blk_e_hbm, blk_act_hbm, nact_hbm,
                  ids_v, hist_v, pos_v, didx_v, blke_v, blka_v, nact_v,
                  xbuf0_v, xbuf1_v, gsem0, gsem1, ssem0, ssem1):
    c = lax.axis_index("c")
    s = lax.axis_index("s")
    wid = c * 16 + s
    base = wid * CHUNK
    cpb = 2 * BT // CHUNK
    kpb = BT // CHUNK
    tok_base = (wid // cpb) * BT + (wid % kpb) * CHUNK
    xbufs = (xbuf0_v, xbuf1_v)
    gsems = (gsem0, gsem1)
    ssems = (ssem0, ssem1)

    def fire_gather(ch):
        start = pl.multiple_of(tok_base + ch * RCH, 8)
        return pltpu.async_copy(x_hbm.at[pl.ds(start, RCH)], xbufs[ch % 2],
                                gsems[ch % 2])

    hg = [None] * NCH
    hg[0] = fire_gather(0)
    hg[1] = fire_gather(1)

    erow = wid // cpb
    ecol = (wid % cpb) * CHUNK
    pltpu.sync_copy(e01_hbm.at[pl.ds(erow, 1), 0, pl.ds(ecol, CHUNK)], ids_v)
    pltpu.sync_copy(hist_hbm, hist_v)
    lane = lax.iota(jnp.int32, L)

    tot = jnp.zeros((L,), jnp.int32)
    pre = jnp.zeros((L,), jnp.int32)
    for m in range(NW // 2):
        hv = hist_v[pl.ds(m * L, L)]
        cidx = jnp.where(lane >= 8, 2 * m + 1, 2 * m)
        tot = tot + hv
        pre = pre + jnp.where(cidx < wid, hv, 0)
    fold_idx = (lane + 8) % 16
    tot = jnp.where(lane < 8, tot + _vgather(tot, fold_idx), 0)
    pre = jnp.where(lane < 8, pre + _vgather(pre, fold_idx), 0)

    padded = ((tot + (B - 1)) >> LOG2B) << LOG2B
    incl = plsc.cumsum(padded)
    excl = incl - padded
    cursors = excl + pre

    for v in range(CHUNK // L):
        ev = ids_v[0, pl.ds(v * L, L)]
        posv = jnp.zeros((L,), jnp.int32)
        for e in range(E):
            mask = ev == e
            cnt = mask.astype(jnp.int32)
            rank = plsc.cumsum(cnt) - cnt
            posv = jnp.where(mask, _splat(cursors, e) + rank, posv)
            pc = plsc.all_reduce_population_count(mask)
            cursors = cursors + jnp.where(lane == e, pc, 0)
        pos_v[pl.ds(v * L, L)] = posv
        didx_v[v // 2, pl.ds((v % 2) * L, L)] = posv
    pltpu.sync_copy(pos_v, pos_hbm.at[pl.ds(base, CHUNK)])

    @pl.when(wid == 0)
    def _():
        np_used = _splat(incl, E - 1)
        for m in range(NBP // L):
            bbase = (lane + m * L) << LOG2B
            acc = jnp.zeros((L,), jnp.int32)
            for e in range(E):
                acc = acc + (_splat(incl, e) <= bbase).astype(jnp.int32)
            blke_v[pl.ds(m * L, L)] = jnp.minimum(acc, E - 1)
            blka_v[pl.ds(m * L, L)] = (bbase < np_used).astype(jnp.int32)
        nact_v[pl.ds(0, L)] = np_used >> LOG2B
        pltpu.sync_copy(blke_v, blk_e_hbm)
        pltpu.sync_copy(blka_v, blk_act_hbm)
        pltpu.sync_copy(nact_v.at[pl.ds(0, 8)], nact_hbm)

    for ch in range(NCH):
        hg[ch].wait()
        hs = pltpu.async_copy(xbufs[ch % 2], xs_hbm.at[didx_v.at[ch]],
                              ssems[ch % 2])
        hs.wait()
        if ch + 2 < NCH:
            hg[ch + 2] = fire_gather(ch + 2)


def _routing(e01, hist, x):
    mesh = plsc.VectorSubcoreMesh(core_axis_name="c", subcore_axis_name="s")
    kfn = functools.partial(
        pl.kernel,
        out_type=[
            jax.ShapeDtypeStruct((NP, D), jnp.float32),
            jax.ShapeDtypeStruct((2 * T,), jnp.int32),
            jax.ShapeDtypeStruct((NBP,), jnp.int32),
            jax.ShapeDtypeStruct((NBP,), jnp.int32),
            jax.ShapeDtypeStruct((8,), jnp.int32),
        ],
        mesh=mesh,
        scratch_types=[
            pltpu.VMEM((1, CHUNK), jnp.int32),
            pltpu.VMEM((NW * E,), jnp.int32),
            pltpu.VMEM((CHUNK,), jnp.int32),
            pltpu.VMEM((NCH, RCH), jnp.int32),
            pltpu.VMEM((NBP,), jnp.int32),
            pltpu.VMEM((NBP,), jnp.int32),
            pltpu.VMEM((L,), jnp.int32),
            pltpu.VMEM((RCH, D), jnp.float32),
            pltpu.VMEM((RCH, D), jnp.float32),
            pltpu.SemaphoreType.DMA,
            pltpu.SemaphoreType.DMA,
            pltpu.SemaphoreType.DMA,
            pltpu.SemaphoreType.DMA,
        ],
        compiler_params=pltpu.CompilerParams(needs_layout_passes=False),
    )(_routing_body)
    return kfn(e01, hist, x)


def _ffn_body(blk_e_ref, blk_act_ref, nact_ref, xs_ref, w1_ref, b1_ref,
              w2_ref, b2_ref, ys_ref):
    b = pl.program_id(0)

    @pl.when(blk_act_ref[b] == 1)
    def _():
        xb = xs_ref[...]
        h = jax.nn.gelu(
            jnp.dot(xb, w1_ref[0], preferred_element_type=jnp.float32)
            + b1_ref[0])
        ys_ref[...] = (
            jnp.dot(h, w2_ref[0], preferred_element_type=jnp.float32)
            + b2_ref[0])


def _ffn(blk_e, blk_act, nact, xs, W1, b1, W2, b2):
    def bc(b, na):
        return jnp.minimum(b, na[0] - 1)

    grid_spec = pltpu.PrefetchScalarGridSpec(
        num_scalar_prefetch=3,
        grid=(NB,),
        in_specs=[
            pl.BlockSpec((B, D), lambda b, be, ba, na: (bc(b, na), 0)),
            pl.BlockSpec((1, D, H),
                         lambda b, be, ba, na: (be[bc(b, na)], 0, 0)),
            pl.BlockSpec((1, 1, H),
                         lambda b, be, ba, na: (be[bc(b, na)], 0, 0)),
            pl.BlockSpec((1, H, D),
                         lambda b, be, ba, na: (be[bc(b, na)], 0, 0)),
            pl.BlockSpec((1, 1, D),
                         lambda b, be, ba, na: (be[bc(b, na)], 0, 0)),
        ],
        out_specs=pl.BlockSpec((B, D), lambda b, be, ba, na: (bc(b, na), 0)),
    )
    return pl.pallas_call(
        _ffn_body,
        grid_spec=grid_spec,
        out_shape=jax.ShapeDtypeStruct((NP, D), jnp.float32),
        compiler_params=pltpu.CompilerParams(
            dimension_semantics=("arbitrary",),
        ),
    )(blk_e, blk_act, nact, xs, W1, b1.reshape(E, 1, H), W2,
      b2.reshape(E, 1, D))


def _combine_body(ys_hbm, pos_hbm, w01_hbm, y_hbm,
                  pidx0_v, pidx1_v, wvec0_v, wvec1_v,
                  b00, b01, b10, b11, ob0, ob1,
                  g00, g01, g10, g11, ss0, ss1):
    c = lax.axis_index("c")
    s = lax.axis_index("s")
    wid = c * 16 + s
    tbase = wid * TPW
    nch = TPW // L
    bufs = ((b00, b01), (b10, b11))
    obufs = (ob0, ob1)
    gsems = ((g00, g01), (g10, g11))
    ssems = (ss0, ss1)

    blk = tbase // BT
    off = tbase % BT
    j0b = pl.multiple_of(blk * 2 * BT + off, 8)
    j1b = pl.multiple_of(j0b + BT, 8)
    pltpu.sync_copy(pos_hbm.at[pl.ds(j0b, TPW)], pidx0_v)
    pltpu.sync_copy(pos_hbm.at[pl.ds(j1b, TPW)], pidx1_v)
    pltpu.sync_copy(w01_hbm.at[pl.ds(blk, 1), 0, pl.ds(off, TPW)], wvec0_v)
    pltpu.sync_copy(w01_hbm.at[pl.ds(blk, 1), 0, pl.ds(off + BT, TPW)],
                    wvec1_v)

    def fire(ch):
        par = ch % 2
        h0 = pltpu.async_copy(ys_hbm.at[pidx0_v.at[pl.ds(ch * L, L)]],
                              bufs[par][0], gsems[par][0])
        h1 = pltpu.async_copy(ys_hbm.at[pidx1_v.at[pl.ds(ch * L, L)]],
                              bufs[par][1], gsems[par][1])
        return h0, h1

    hg = [None] * nch
    hs = [None] * nch
    hg[0] = fire(0)
    hg[1] = fire(1)
    for ch in range(nch):
        par = ch % 2
        hg[ch][0].wait()
        hg[ch][1].wait()
        if ch >= 2:
            hs[ch - 2].wait()
        b0, b1 = bufs[par]
        ob = obufs[par]
        wv0 = wvec0_v[0, pl.ds(ch * L, L)]
        wv1 = wvec1_v[0, pl.ds(ch * L, L)]
        for half in range(2):
            w0s = [_splat(wv0, half * 8 + r) for r in range(8)]
            w1s = [_splat(wv1, half * 8 + r) for r in range(8)]

            def jbody(j, _, half=half, w0s=w0s, w1s=w1s):
                for r in range(8):
                    rr = half * 8 + r
                    a = b0[rr, pl.ds(j * L, L)]
                    bb = b1[rr, pl.ds(j * L, L)]
                    ob[rr, pl.ds(j * L, L)] = w0s[r] * a + w1s[r] * bb
                return 0

            lax.fori_loop(0, D // L, jbody, 0, unroll=2)
        t0 = pl.multiple_of(tbase + ch * L, 8)
        hs[ch] = pltpu.async_copy(ob, y_hbm.at[pl.ds(t0, L)], ssems[par])
        if ch + 2 < nch:
            hg[ch + 2] = fire(ch + 2)
    hs[nch - 2].wait()
    hs[nch - 1].wait()


def _combine(ys, pos, w01f):
    mesh = plsc.VectorSubcoreMesh(core_axis_name="c", subcore_axis_name="s")
    kfn = functools.partial(
        pl.kernel,
        out_type=jax.ShapeDtypeStruct((T, D), jnp.float32),
        mesh=mesh,
        scratch_types=(
            [pltpu.VMEM((TPW,), jnp.int32)] * 2
            + [pltpu.VMEM((1, TPW), jnp.float32)] * 2
            + [pltpu.VMEM((L, D), jnp.float32)] * 6
            + [pltpu.SemaphoreType.DMA] * 6
        ),
        compiler_params=pltpu.CompilerParams(needs_layout_passes=False),
    )(_combine_body)
    return kfn(ys, pos, w01f)


def kernel(x, Wg, W1, b1, W2, b2):
    e01, w01, hist = _gating(x, Wg)
    xs, pos, blk_e, blk_act, nact = _routing(
        e01, hist.reshape(2 * T // 256 * E), x)
    ys = _ffn(blk_e, blk_act, nact, xs, W1, b1, W2, b2)
    return _combine(ys, pos, w01)


# speedup vs baseline: 1.0333x; 1.0333x over previous
import functools
import jax
import jax.numpy as jnp
from jax import lax
from jax.experimental import pallas as pl
from jax.experimental.pallas import tpu as pltpu
from jax.experimental.pallas import tpu_sc as plsc

T = 4096
D = 1024
H = 1024
E = 8
K = 2

BT = 1024
B = 256
LOG2B = 8
NB = 40
NBP = 48
NP = NB * B
NW = 32
CHUNK = 2 * T // NW
L = 16
TPW = T // NW


def _gating_body(x_ref, wg_ref, e01_ref, w01_ref, hist_ref):
    x = x_ref[...]
    logits = jnp.dot(x, wg_ref[...], preferred_element_type=jnp.float32)
    ids = lax.broadcasted_iota(jnp.int32, logits.shape, 1)
    m1 = jnp.max(logits, axis=1, keepdims=True)
    i1 = jnp.min(jnp.where(logits == m1, ids, E), axis=1, keepdims=True)
    masked = jnp.where(ids == i1, -jnp.inf, logits)
    m2 = jnp.max(masked, axis=1, keepdims=True)
    i2 = jnp.min(jnp.where(masked == m2, ids, E), axis=1, keepdims=True)
    s = jnp.exp(m2 - m1)
    w1g = 1.0 / (1.0 + s)
    w2g = 1.0 - w1g
    e01_ref[...] = lax.transpose(
        jnp.concatenate([i1, i2], axis=0), (1, 0)).reshape(1, 1, 2 * BT)
    w01_ref[...] = lax.transpose(
        jnp.concatenate([w1g, w2g], axis=0), (1, 0)).reshape(1, 1, 2 * BT)
    oh0 = (i1 == ids).astype(jnp.int32)
    oh1 = (i2 == ids).astype(jnp.int32)
    subs = [jnp.sum(oh0[ss * 256:(ss + 1) * 256], axis=0)
            for ss in range(BT // 256)]
    subs += [jnp.sum(oh1[ss * 256:(ss + 1) * 256], axis=0)
             for ss in range(BT // 256)]
    hist_ref[...] = jnp.concatenate(subs).reshape(1, 1, 2 * BT // 256 * E)


def _gating(x, Wg):
    nblk = T // BT
    hlane = 2 * BT // 256 * E
    return pl.pallas_call(
        _gating_body,
        grid=(nblk,),
        in_specs=[
            pl.BlockSpec((BT, D), lambda i: (i, 0)),
            pl.BlockSpec((D, E), lambda i: (0, 0)),
        ],
        out_specs=[
            pl.BlockSpec((1, 1, 2 * BT), lambda i: (i, 0, 0)),
            pl.BlockSpec((1, 1, 2 * BT), lambda i: (i, 0, 0)),
            pl.BlockSpec((1, 1, hlane), lambda i: (i, 0, 0)),
        ],
        out_shape=[
            jax.ShapeDtypeStruct((nblk, 1, 2 * BT), jnp.int32),
            jax.ShapeDtypeStruct((nblk, 1, 2 * BT), jnp.float32),
            jax.ShapeDtypeStruct((nblk, 1, hlane), jnp.int32),
        ],
        compiler_params=pltpu.CompilerParams(
            dimension_semantics=("parallel",),
        ),
    )(x, Wg)


def _vgather(vec, idx):
    return lax.gather(
        vec, idx[:, None],
        lax.GatherDimensionNumbers(
            offset_dims=(), collapsed_slice_dims=(0,), start_index_map=(0,)),
        (1,), mode=lax.GatherScatterMode.PROMISE_IN_BOUNDS)


def _splat(vec, e):
    return _vgather(vec, jnp.full((L,), e, jnp.int32))


RCH = 32
NCH = CHUNK // RCH


def _routing_body(e01_hbm, hist_hbm, x_hbm,
                  xs_hbm, pos_hbm, blk_e_hbm, blk_act_hbm, nact_hbm,
                  ids_v, hist_v, pos_v, didx_v, blke_v, blka_v, nact_v,
                  xbuf0_v, xbuf1_v, gsem0, gsem1, ssem0, ssem1):
    c = lax.axis_index("c")
    s = lax.axis_index("s")
    wid = c * 16 + s
    base = wid * CHUNK
    cpb = 2 * BT // CHUNK
    kpb = BT // CHUNK
    tok_base = (wid // cpb) * BT + (wid % kpb) * CHUNK
    xbufs = (xbuf0_v, xbuf1_v)
    gsems = (gsem0, gsem1)
    ssems = (ssem0, ssem1)

    def fire_gather(ch):
        start = pl.multiple_of(tok_base + ch * RCH, 8)
        return pltpu.async_copy(x_hbm.at[pl.ds(start, RCH)], xbufs[ch % 2],
                                gsems[ch % 2])

    hg = [None] * NCH
    hg[0] = fire_gather(0)
    hg[1] = fire_gather(1)

    erow = wid // cpb
    ecol = (wid % cpb) * CHUNK
    pltpu.sync_copy(e01_hbm.at[pl.ds(erow, 1), 0, pl.ds(ecol, CHUNK)], ids_v)
    pltpu.sync_copy(hist_hbm, hist_v)
    lane = lax.iota(jnp.int32, L)

    tot = jnp.zeros((L,), jnp.int32)
    pre = jnp.zeros((L,), jnp.int32)
    for m in range(NW // 2):
        hv = hist_v[pl.ds(m * L, L)]
        cidx = jnp.where(lane >= 8, 2 * m + 1, 2 * m)
        tot = tot + hv
        pre = pre + jnp.where(cidx < wid, hv, 0)
    fold_idx = (lane + 8) % 16
    tot = jnp.where(lane < 8, tot + _vgather(tot, fold_idx), 0)
    pre = jnp.where(lane < 8, pre + _vgather(pre, fold_idx), 0)

    padded = ((tot + (B - 1)) >> LOG2B) << LOG2B
    incl = plsc.cumsum(padded)
    excl = incl - padded
    cursors = excl + pre

    for v in range(CHUNK // L):
        ev = ids_v[0, pl.ds(v * L, L)]
        posv = jnp.zeros((L,), jnp.int32)
        for e in range(E):
            mask = ev == e
            cnt = mask.astype(jnp.int32)
            rank = plsc.cumsum(cnt) - cnt
            posv = jnp.where(mask, _splat(cursors, e) + rank, posv)
            pc = plsc.all_reduce_population_count(mask)
            cursors = cursors + jnp.where(lane == e, pc, 0)
        pos_v[pl.ds(v * L, L)] = posv
        didx_v[v // 2, pl.ds((v % 2) * L, L)] = posv
    pltpu.sync_copy(pos_v, pos_hbm.at[pl.ds(base, CHUNK)])

    @pl.when(wid == 0)
    def _():
        np_used = _splat(incl, E - 1)
        for m in range(NBP // L):
            bbase = (lane + m * L) << LOG2B
            acc = jnp.zeros((L,), jnp.int32)
            for e in range(E):
                acc = acc + (_splat(incl, e) <= bbase).astype(jnp.int32)
            blke_v[pl.ds(m * L, L)] = jnp.minimum(acc, E - 1)
            blka_v[pl.ds(m * L, L)] = (bbase < np_used).astype(jnp.int32)
        nact_v[pl.ds(0, L)] = np_used >> LOG2B
        pltpu.sync_copy(blke_v, blk_e_hbm)
        pltpu.sync_copy(blka_v, blk_act_hbm)
        pltpu.sync_copy(nact_v.at[pl.ds(0, 8)], nact_hbm)

    for ch in range(NCH):
        hg[ch].wait()
        hs = pltpu.async_copy(xbufs[ch % 2], xs_hbm.at[didx_v.at[ch]],
                              ssems[ch % 2])
        hs.wait()
        if ch + 2 < NCH:
            hg[ch + 2] = fire_gather(ch + 2)


def _routing(e01, hist, x):
    mesh = plsc.VectorSubcoreMesh(core_axis_name="c", subcore_axis_name="s")
    kfn = functools.partial(
        pl.kernel,
        out_type=[
            jax.ShapeDtypeStruct((NP, D), jnp.float32),
            jax.ShapeDtypeStruct((2 * T,), jnp.int32),
            jax.ShapeDtypeStruct((NBP,), jnp.int32),
            jax.ShapeDtypeStruct((NBP,), jnp.int32),
            jax.ShapeDtypeStruct((8,), jnp.int32),
        ],
        mesh=mesh,
        scratch_types=[
            pltpu.VMEM((1, CHUNK), jnp.int32),
            pltpu.VMEM((NW * E,), jnp.int32),
            pltpu.VMEM((CHUNK,), jnp.int32),
            pltpu.VMEM((NCH, RCH), jnp.int32),
            pltpu.VMEM((NBP,), jnp.int32),
            pltpu.VMEM((NBP,), jnp.int32),
            pltpu.VMEM((L,), jnp.int32),
            pltpu.VMEM((RCH, D), jnp.float32),
            pltpu.VMEM((RCH, D), jnp.float32),
            pltpu.SemaphoreType.DMA,
            pltpu.SemaphoreType.DMA,
            pltpu.SemaphoreType.DMA,
            pltpu.SemaphoreType.DMA,
        ],
        compiler_params=pltpu.CompilerParams(needs_layout_passes=False),
    )(_routing_body)
    return kfn(e01, hist, x)


def _ffn_body(blk_e_ref, blk_act_ref, nact_ref, xs_ref, w1_ref, b1_ref,
              w2_ref, b2_ref, ys_ref):
    b = pl.program_id(0)

    @pl.when(blk_act_ref[b] == 1)
    def _():
        xb = xs_ref[...]
        h = jax.nn.gelu(
            jnp.dot(xb, w1_ref[0], preferred_element_type=jnp.float32)
            + b1_ref[0])
        ys_ref[...] = (
            jnp.dot(h, w2_ref[0], preferred_element_type=jnp.float32)
            + b2_ref[0])


def _ffn(blk_e, blk_act, nact, xs, W1, b1, W2, b2):
    def bc(b, na):
        return jnp.minimum(b, na[0] - 1)

    grid_spec = pltpu.PrefetchScalarGridSpec(
        num_scalar_prefetch=3,
        grid=(NB,),
        in_specs=[
            pl.BlockSpec((B, D), lambda b, be, ba, na: (bc(b, na), 0)),
            pl.BlockSpec((1, D, H),
                         lambda b, be, ba, na: (be[bc(b, na)], 0, 0)),
            pl.BlockSpec((1, 1, H),
                         lambda b, be, ba, na: (be[bc(b, na)], 0, 0)),
            pl.BlockSpec((1, H, D),
                         lambda b, be, ba, na: (be[bc(b, na)], 0, 0)),
            pl.BlockSpec((1, 1, D),
                         lambda b, be, ba, na: (be[bc(b, na)], 0, 0)),
        ],
        out_specs=pl.BlockSpec((B, D), lambda b, be, ba, na: (bc(b, na), 0)),
    )
    return pl.pallas_call(
        _ffn_body,
        grid_spec=grid_spec,
        out_shape=jax.ShapeDtypeStruct((NP, D), jnp.float32),
        compiler_params=pltpu.CompilerParams(
            dimension_semantics=("arbitrary",),
        ),
    )(blk_e, blk_act, nact, xs, W1, b1.reshape(E, 1, H), W2,
      b2.reshape(E, 1, D))


def _combine_body(ys_hbm, pos_hbm, w01_hbm, y_hbm,
                  pidx0_v, pidx1_v, wvec0_v, wvec1_v,
                  b00, b01, b10, b11, ob0, ob1,
                  g00, g01, g10, g11, ss0, ss1):
    c = lax.axis_index("c")
    s = lax.axis_index("s")
    wid = c * 16 + s
    tbase = wid * TPW
    nch = TPW // L
    bufs = ((b00, b01), (b10, b11))
    obufs = (ob0, ob1)
    gsems = ((g00, g01), (g10, g11))
    ssems = (ss0, ss1)

    blk = tbase // BT
    off = tbase % BT
    j0b = pl.multiple_of(blk * 2 * BT + off, 8)
    j1b = pl.multiple_of(j0b + BT, 8)
    pltpu.sync_copy(pos_hbm.at[pl.ds(j0b, TPW)], pidx0_v)
    pltpu.sync_copy(pos_hbm.at[pl.ds(j1b, TPW)], pidx1_v)
    pltpu.sync_copy(w01_hbm.at[pl.ds(blk, 1), 0, pl.ds(off, TPW)], wvec0_v)
    pltpu.sync_copy(w01_hbm.at[pl.ds(blk, 1), 0, pl.ds(off + BT, TPW)],
                    wvec1_v)

    def fire(ch):
        par = ch % 2
        h0 = pltpu.async_copy(ys_hbm.at[pidx0_v.at[pl.ds(ch * L, L)]],
                              bufs[par][0], gsems[par][0])
        h1 = pltpu.async_copy(ys_hbm.at[pidx1_v.at[pl.ds(ch * L, L)]],
                              bufs[par][1], gsems[par][1])
        return h0, h1

    hg = [None] * nch
    hs = [None] * nch
    hg[0] = fire(0)
    hg[1] = fire(1)
    for ch in range(nch):
        par = ch % 2
        hg[ch][0].wait()
        hg[ch][1].wait()
        if ch >= 2:
            hs[ch - 2].wait()
        b0, b1 = bufs[par]
        ob = obufs[par]
        wv0 = wvec0_v[0, pl.ds(ch * L, L)]
        wv1 = wvec1_v[0, pl.ds(ch * L, L)]
        for half in range(2):
            w0s = [_splat(wv0, half * 8 + r) for r in range(8)]
            w1s = [_splat(wv1, half * 8 + r) for r in range(8)]

            def jbody(j, _, half=half, w0s=w0s, w1s=w1s):
                for r in range(8):
                    rr = half * 8 + r
                    a = b0[rr, pl.ds(j * L, L)]
                    bb = b1[rr, pl.ds(j * L, L)]
                    ob[rr, pl.ds(j * L, L)] = w0s[r] * a + w1s[r] * bb
                return 0

            lax.fori_loop(0, D // L, jbody, 0)
        t0 = pl.multiple_of(tbase + ch * L, 8)
        hs[ch] = pltpu.async_copy(ob, y_hbm.at[pl.ds(t0, L)], ssems[par])
        if ch + 2 < nch:
            hg[ch + 2] = fire(ch + 2)
    hs[nch - 2].wait()
    hs[nch - 1].wait()


def _combine(ys, pos, w01f):
    mesh = plsc.VectorSubcoreMesh(core_axis_name="c", subcore_axis_name="s")
    kfn = functools.partial(
        pl.kernel,
        out_type=jax.ShapeDtypeStruct((T, D), jnp.float32),
        mesh=mesh,
        scratch_types=(
            [pltpu.VMEM((TPW,), jnp.int32)] * 2
            + [pltpu.VMEM((1, TPW), jnp.float32)] * 2
            + [pltpu.VMEM((L, D), jnp.float32)] * 6
            + [pltpu.SemaphoreType.DMA] * 6
        ),
        compiler_params=pltpu.CompilerParams(needs_layout_passes=False),
    )(_combine_body)
    return kfn(ys, pos, w01f)


def kernel(x, Wg, W1, b1, W2, b2):
    e01, w01, hist = _gating(x, Wg)
    xs, pos, blk_e, blk_act, nact = _routing(
        e01, hist.reshape(2 * T // 256 * E), x)
    ys = _ffn(blk_e, blk_act, nact, xs, W1, b1, W2, b2)
    return _combine(ys, pos, w01)
